# baseline (speedup 1.0000x reference)
import functools

import jax
import jax.numpy as jnp
import numpy as np
from jax.experimental import pallas as pl
from jax.experimental.pallas import tpu as pltpu

N = 32768
K = 32
R0 = 0.2
R1 = 0.5
M1 = N // K
M2 = M1 // K
D0 = 64
D1 = 128


def _fps_kernel(px_ref, py_ref, pz_ref,
                p1x_ref, p1y_ref, p1z_ref, p2x_ref, p2y_ref, p2z_ref,
                mind_ref):
    rows = jax.lax.broadcasted_iota(jnp.int32, (256, 128), 0)
    cols = jax.lax.broadcasted_iota(jnp.int32, (256, 128), 1)
    iof = rows * 128 + cols
    r1 = jax.lax.broadcasted_iota(jnp.int32, (8, 128), 0)
    c1 = jax.lax.broadcasted_iota(jnp.int32, (8, 128), 1)
    io1 = r1 * 128 + c1

    px = px_ref[...]
    py = py_ref[...]
    pz = pz_ref[...]

    def _sel_coord(arr, oh):
        return jnp.sum(jnp.where(oh, arr, 0.0))

    oh0 = iof == 0
    qx = _sel_coord(px, oh0)
    qy = _sel_coord(py, oh0)
    qz = _sel_coord(pz, oh0)
    dx = px - qx
    dy = py - qy
    dz = pz - qz
    mind_ref[...] = (dx * dx + dy * dy) + dz * dz
    ohw = io1 == 0
    p1x_ref[...] = jnp.where(ohw, qx, 0.0)
    p1y_ref[...] = jnp.where(ohw, qy, 0.0)
    p1z_ref[...] = jnp.where(ohw, qz, 0.0)

    def body(i, _):
        mind = mind_ref[...]
        m = jnp.max(mind)
        nxt = jnp.min(jnp.where(mind == m, iof, (2**30)))
        oh = iof == nxt
        qx = _sel_coord(px, oh)
        qy = _sel_coord(py, oh)
        qz = _sel_coord(pz, oh)
        dx = px - qx
        dy = py - qy
        dz = pz - qz
        d = (dx * dx + dy * dy) + dz * dz
        mind_ref[...] = jnp.minimum(mind, d)
        ohw = io1 == i
        p1x_ref[...] = jnp.where(ohw, qx, p1x_ref[...])
        p1y_ref[...] = jnp.where(ohw, qy, p1y_ref[...])
        p1z_ref[...] = jnp.where(ohw, qz, p1z_ref[...])
        return 0

    jax.lax.fori_loop(1, M1, body, 0)

    gx = p1x_ref[...]
    gy = p1y_ref[...]
    gz = p1z_ref[...]
    oh0b = io1 == 0
    qx = _sel_coord(gx, oh0b)
    qy = _sel_coord(gy, oh0b)
    qz = _sel_coord(gz, oh0b)
    dx = gx - qx
    dy = gy - qy
    dz = gz - qz
    mind2 = (dx * dx + dy * dy) + dz * dz
    p2x_ref[...] = jnp.where(oh0b, qx, 0.0)
    p2y_ref[...] = jnp.where(oh0b, qy, 0.0)
    p2z_ref[...] = jnp.where(oh0b, qz, 0.0)

    def body2(i, mind2):
        m = jnp.max(mind2)
        nxt = jnp.min(jnp.where(mind2 == m, io1, (2**30)))
        oh = io1 == nxt
        qx = _sel_coord(gx, oh)
        qy = _sel_coord(gy, oh)
        qz = _sel_coord(gz, oh)
        dx = gx - qx
        dy = gy - qy
        dz = gz - qz
        d = (dx * dx + dy * dy) + dz * dz
        ohw = io1 == i
        p2x_ref[...] = jnp.where(ohw, qx, p2x_ref[...])
        p2y_ref[...] = jnp.where(ohw, qy, p2y_ref[...])
        p2z_ref[...] = jnp.where(ohw, qz, p2z_ref[...])
        return jnp.minimum(mind2, d)

    jax.lax.fori_loop(1, M2, body2, mind2)


def _fps_pallas(points):
    px = points[:, 0].reshape(256, 128)
    py = points[:, 1].reshape(256, 128)
    pz = points[:, 2].reshape(256, 128)
    shp = jax.ShapeDtypeStruct((8, 128), jnp.float32)
    outs = pl.pallas_call(
        _fps_kernel,
        out_shape=(shp,) * 6,
        scratch_shapes=[pltpu.VMEM((256, 128), jnp.float32)],
    )(px, py, pz)
    p1 = jnp.stack([o.reshape(M1) for o in outs[:3]], axis=1)
    p2 = jnp.stack([o.reshape(M1)[:M2] for o in outs[3:]], axis=1)
    return p1, p2


def _d2_kernel(y_ref, x_ref, o_ref):
    y = y_ref[...]
    x = x_ref[...]
    yy = y[:, 0:1] * y[:, 0:1] + y[:, 1:2] * y[:, 1:2] + y[:, 2:3] * y[:, 2:3]
    xx = x[:, 0:1] * x[:, 0:1] + x[:, 1:2] * x[:, 1:2] + x[:, 2:3] * x[:, 2:3]
    m = jax.lax.dot_general(y, x, (((1,), (1,)), ((), ())),
                            preferred_element_type=jnp.float32)
    o_ref[...] = (yy + xx.T) - 2.0 * m


def _d2_pallas(y, x):
    My, Nx = y.shape[0], x.shape[0]
    QB = min(My, 256)
    CB = min(Nx, 4096)
    return pl.pallas_call(
        _d2_kernel,
        grid=(My // QB, Nx // CB),
        in_specs=[
            pl.BlockSpec((QB, 3), lambda i, j: (i, 0)),
            pl.BlockSpec((CB, 3), lambda i, j: (j, 0)),
        ],
        out_specs=pl.BlockSpec((QB, CB), lambda i, j: (i, j)),
        out_shape=jax.ShapeDtypeStruct((My, Nx), jnp.float32),
    )(y, x)


def _sc_gather_points(px, py, pz, idx):
    from jax.experimental.pallas import tpu_sc as plsc
    B = idx.shape[0]
    info = plsc.get_sparse_core_info()
    NW = info.num_cores * info.num_subcores
    bpw = B // NW
    mesh = plsc.VectorSubcoreMesh(core_axis_name="c", subcore_axis_name="s")

    @functools.partial(
        pl.kernel, mesh=mesh,
        out_type=[jax.ShapeDtypeStruct((B,), jnp.float32)] * 3,
        scratch_types=[
            pltpu.VMEM((bpw,), jnp.int32),
            pltpu.VMEM((bpw,), jnp.float32),
            pltpu.VMEM((bpw,), jnp.float32),
            pltpu.VMEM((bpw,), jnp.float32),
            pltpu.SemaphoreType.DMA,
        ],
    )
    def k(px_hbm, py_hbm, pz_hbm, idx_hbm, ox_hbm, oy_hbm, oz_hbm,
          idx_v, xv, yv, zv, sem):
        wid = jax.lax.axis_index("s") * info.num_cores + jax.lax.axis_index("c")
        base = wid * bpw
        pltpu.sync_copy(idx_hbm.at[pl.ds(base, bpw)], idx_v)
        pltpu.async_copy(px_hbm.at[idx_v], xv, sem).wait()
        pltpu.async_copy(py_hbm.at[idx_v], yv, sem).wait()
        pltpu.async_copy(pz_hbm.at[idx_v], zv, sem).wait()
        pltpu.sync_copy(xv, ox_hbm.at[pl.ds(base, bpw)])
        pltpu.sync_copy(yv, oy_hbm.at[pl.ds(base, bpw)])
        pltpu.sync_copy(zv, oz_hbm.at[pl.ds(base, bpw)])

    ox, oy, oz = k(px, py, pz, idx)
    return jnp.stack([ox, oy, oz], axis=1)


def _knn_radius(x, y, r, k):
    d2 = (jnp.sum(y * y, axis=1)[:, None] + jnp.sum(x * x, axis=1)[None, :]
          - 2.0 * (y @ x.T))
    neg, idx = jax.lax.top_k(-d2, k)
    valid = (-neg) <= r * r
    return idx, valid


_QB0 = 8
_NR = N // 128


def _knn0_kernel(y_ref, xt_ref, idx_ref, mval_ref, d2_ref):
    y = y_ref[...]
    xt = xt_ref[...]
    yy = y[:, 0:1] * y[:, 0:1] + y[:, 1:2] * y[:, 1:2] + y[:, 2:3] * y[:, 2:3]
    xx = xt[0:1, :] * xt[0:1, :] + xt[1:2, :] * xt[1:2, :] + xt[2:3, :] * xt[2:3, :]
    mm = jnp.dot(y, xt, preferred_element_type=jnp.float32)
    d2 = ((yy + xx) - 2.0 * mm).reshape(_QB0, _NR, 128)
    d2_ref[...] = d2
    bm = jnp.min(d2, axis=2)
    io_bm = jax.lax.broadcasted_iota(jnp.int32, (_QB0, _NR), 1)
    io32 = jax.lax.broadcasted_iota(jnp.int32, (_QB0, K), 1)

    def bstep(kk, carry):
        curv, curi, sb = carry
        elig = (bm > curv) | ((bm == curv) & (io_bm > curi))
        bmm = jnp.where(elig, bm, float("inf"))
        m2 = jnp.min(bmm, axis=1, keepdims=True)
        b2 = jnp.min(jnp.where(elig & (bm == m2), io_bm, 2**30),
                     axis=1, keepdims=True)
        sb = jnp.where(io32 == kk, b2, sb)
        return m2, b2, sb

    _, _, sb = jax.lax.fori_loop(
        0, K, bstep,
        (jnp.full((_QB0, 1), -float("inf"), jnp.float32),
         jnp.full((_QB0, 1), -1, jnp.int32),
         jnp.zeros((_QB0, K), jnp.int32)))

    iofr = (jax.lax.broadcasted_iota(jnp.int32, (_NR, 128), 0) * 128
            + jax.lax.broadcasted_iota(jnp.int32, (_NR, 128), 1)
            ).astype(jnp.float32)
    cds = []
    cis = []
    for q in range(_QB0):
        sel = (sb[q:q + 1, :].reshape(K, 1) ==
               jax.lax.broadcasted_iota(jnp.int32, (K, _NR), 1))
        self32 = sel.astype(jnp.float32)
        cds.append(jnp.dot(self32, d2_ref[q],
                           preferred_element_type=jnp.float32,
                           precision=jax.lax.Precision.HIGHEST))
        cis.append(jnp.dot(self32, iofr,
                           preferred_element_type=jnp.float32,
                           precision=jax.lax.Precision.HIGHEST))
    cd = jnp.stack(cds)
    ci = jnp.stack(cis).astype(jnp.int32)

    def step(kk, carry):
        curv, curi, idxa, mva = carry
        cv = curv[:, :, None]
        cidx = curi[:, :, None]
        elig = (cd > cv) | ((cd == cv) & (ci > cidx))
        cdm = jnp.where(elig, cd, float("inf"))
        m = jnp.min(cdm, axis=(1, 2))[:, None]
        fl = jnp.min(jnp.where(elig & (cd == m[:, :, None]), ci, 2**30),
                     axis=(1, 2))[:, None]
        idxa = jnp.where(io32 == kk, fl, idxa)
        mva = jnp.where(io32 == kk, m, mva)
        return m, fl, idxa, mva

    _, _, idxa, mva = jax.lax.fori_loop(
        0, K, step,
        (jnp.full((_QB0, 1), -float("inf"), jnp.float32),
         jnp.full((_QB0, 1), -1, jnp.int32),
         jnp.zeros((_QB0, K), jnp.int32),
         jnp.zeros((_QB0, K), jnp.float32)))
    idx_ref[...] = idxa
    mval_ref[...] = mva


def _knn0_pallas(points, p1, r):
    xt = points.T
    idx, mval = pl.pallas_call(
        _knn0_kernel,
        grid=(M1 // _QB0,),
        in_specs=[
            pl.BlockSpec((_QB0, 3), lambda i: (i, 0)),
            pl.BlockSpec((3, N), lambda i: (0, 0)),
        ],
        out_specs=[
            pl.BlockSpec((_QB0, K), lambda i: (i, 0)),
            pl.BlockSpec((_QB0, K), lambda i: (i, 0)),
        ],
        out_shape=[
            jax.ShapeDtypeStruct((M1, K), jnp.int32),
            jax.ShapeDtypeStruct((M1, K), jnp.float32),
        ],
        scratch_shapes=[
            pltpu.VMEM((_QB0, _NR, 128), jnp.float32),
        ],
    )(p1, xt)
    valid = mval <= r * r
    return idx, valid


def _decode_kernel(feat_ref, W_ref, b_ref, out2_ref, o_ref):
    d1 = jnp.tanh(feat_ref[...] @ W_ref[...] + b_ref[...])
    o_ref[...] = jnp.tile(out2_ref[...], (1, K)) + d1 * R0


def kernel(points, batch, enc0_W, enc0_b, enc1_W, enc1_b, dec0_W, dec0_b, dec1_W, dec1_b):
    p1, p2 = _fps_pallas(points)
    idx0, valid0 = _knn0_pallas(points, p1, R0)
    rel0 = jnp.where(valid0[..., None], (points[idx0] - p1[:, None, :]) / R0, 0.0)
    h0 = jax.nn.relu(rel0.reshape(-1, 3) @ enc0_W + enc0_b)
    h0 = jnp.where(valid0.reshape(-1, 1), h0, 0.0)
    f1 = h0.reshape(M1, K, D0).max(axis=1)
    idx1, valid1 = _knn_radius(p1, p2, R1, K)
    rel1 = jnp.where(valid1[..., None], (p1[idx1] - p2[:, None, :]) / R1, 0.0)
    g1 = jnp.where(valid1[..., None], f1[idx1], 0.0)
    h1 = jax.nn.relu(jnp.concatenate([rel1, g1], axis=-1).reshape(-1, 3 + D0) @ enc1_W + enc1_b)
    h1 = jnp.where(valid1.reshape(-1, 1), h1, 0.0)
    f2 = h1.reshape(M2, K, D1).max(axis=1)
    cur = idx1.reshape(-1)
    input_points1 = p1[cur]
    nxt = idx0[cur].reshape(-1)
    input_points0 = _sc_gather_points(
        points[:, 0].copy(), points[:, 1].copy(), points[:, 2].copy(), nxt)
    d0 = (f2 @ dec0_W + dec0_b).reshape(M2, K, 3 + D0)
    rel_a = jnp.tanh(d0[..., :3]).reshape(M2 * K, 3)
    feat_a = jax.nn.relu(d0[..., 3:]).reshape(M2 * K, D0)
    out2 = jnp.repeat(p2, K, axis=0) + rel_a * R1
    out3 = pl.pallas_call(
        _decode_kernel,
        out_shape=jax.ShapeDtypeStruct((M2 * K, K * 3), jnp.float32),
    )(feat_a, dec1_W, dec1_b.reshape(1, K * 3), out2).reshape(M2 * K * K, 3)
    return (out3, f2, input_points0, input_points1)


# speedup vs baseline: 1.0027x; 1.0027x over previous
import functools

import jax
import jax.numpy as jnp
import numpy as np
from jax.experimental import pallas as pl
from jax.experimental.pallas import tpu as pltpu

N = 32768
K = 32
R0 = 0.2
R1 = 0.5
M1 = N // K
M2 = M1 // K
D0 = 64
D1 = 128


def _fps_kernel(px_ref, py_ref, pz_ref,
                p1x_ref, p1y_ref, p1z_ref, p2x_ref, p2y_ref, p2z_ref,
                mind_ref):
    rows = jax.lax.broadcasted_iota(jnp.int32, (256, 128), 0)
    cols = jax.lax.broadcasted_iota(jnp.int32, (256, 128), 1)
    iof = rows * 128 + cols
    r1 = jax.lax.broadcasted_iota(jnp.int32, (8, 128), 0)
    c1 = jax.lax.broadcasted_iota(jnp.int32, (8, 128), 1)
    io1 = r1 * 128 + c1

    px = px_ref[...]
    py = py_ref[...]
    pz = pz_ref[...]

    def _sel_coord(arr, oh):
        return jnp.sum(jnp.where(oh, arr, 0.0))

    oh0 = iof == 0
    qx = _sel_coord(px, oh0)
    qy = _sel_coord(py, oh0)
    qz = _sel_coord(pz, oh0)
    dx = px - qx
    dy = py - qy
    dz = pz - qz
    mind_ref[...] = (dx * dx + dy * dy) + dz * dz
    ohw = io1 == 0
    p1x_ref[...] = jnp.where(ohw, qx, 0.0)
    p1y_ref[...] = jnp.where(ohw, qy, 0.0)
    p1z_ref[...] = jnp.where(ohw, qz, 0.0)

    def body(i, _):
        mind = mind_ref[...]
        m = jnp.max(mind)
        nxt = jnp.min(jnp.where(mind == m, iof, (2**30)))
        oh = iof == nxt
        qx = _sel_coord(px, oh)
        qy = _sel_coord(py, oh)
        qz = _sel_coord(pz, oh)
        dx = px - qx
        dy = py - qy
        dz = pz - qz
        d = (dx * dx + dy * dy) + dz * dz
        mind_ref[...] = jnp.minimum(mind, d)
        ohw = io1 == i
        p1x_ref[...] = jnp.where(ohw, qx, p1x_ref[...])
        p1y_ref[...] = jnp.where(ohw, qy, p1y_ref[...])
        p1z_ref[...] = jnp.where(ohw, qz, p1z_ref[...])
        return 0

    jax.lax.fori_loop(1, M1, body, 0)

    gx = p1x_ref[...]
    gy = p1y_ref[...]
    gz = p1z_ref[...]
    oh0b = io1 == 0
    qx = _sel_coord(gx, oh0b)
    qy = _sel_coord(gy, oh0b)
    qz = _sel_coord(gz, oh0b)
    dx = gx - qx
    dy = gy - qy
    dz = gz - qz
    mind2 = (dx * dx + dy * dy) + dz * dz
    p2x_ref[...] = jnp.where(oh0b, qx, 0.0)
    p2y_ref[...] = jnp.where(oh0b, qy, 0.0)
    p2z_ref[...] = jnp.where(oh0b, qz, 0.0)

    def body2(i, mind2):
        m = jnp.max(mind2)
        nxt = jnp.min(jnp.where(mind2 == m, io1, (2**30)))
        oh = io1 == nxt
        qx = _sel_coord(gx, oh)
        qy = _sel_coord(gy, oh)
        qz = _sel_coord(gz, oh)
        dx = gx - qx
        dy = gy - qy
        dz = gz - qz
        d = (dx * dx + dy * dy) + dz * dz
        ohw = io1 == i
        p2x_ref[...] = jnp.where(ohw, qx, p2x_ref[...])
        p2y_ref[...] = jnp.where(ohw, qy, p2y_ref[...])
        p2z_ref[...] = jnp.where(ohw, qz, p2z_ref[...])
        return jnp.minimum(mind2, d)

    jax.lax.fori_loop(1, M2, body2, mind2)


def _fps_pallas(points):
    px = points[:, 0].reshape(256, 128)
    py = points[:, 1].reshape(256, 128)
    pz = points[:, 2].reshape(256, 128)
    shp = jax.ShapeDtypeStruct((8, 128), jnp.float32)
    outs = pl.pallas_call(
        _fps_kernel,
        out_shape=(shp,) * 6,
        scratch_shapes=[pltpu.VMEM((256, 128), jnp.float32)],
    )(px, py, pz)
    p1 = jnp.stack([o.reshape(M1) for o in outs[:3]], axis=1)
    p2 = jnp.stack([o.reshape(M1)[:M2] for o in outs[3:]], axis=1)
    return p1, p2


def _d2_kernel(y_ref, x_ref, o_ref):
    y = y_ref[...]
    x = x_ref[...]
    yy = y[:, 0:1] * y[:, 0:1] + y[:, 1:2] * y[:, 1:2] + y[:, 2:3] * y[:, 2:3]
    xx = x[:, 0:1] * x[:, 0:1] + x[:, 1:2] * x[:, 1:2] + x[:, 2:3] * x[:, 2:3]
    m = jax.lax.dot_general(y, x, (((1,), (1,)), ((), ())),
                            preferred_element_type=jnp.float32)
    o_ref[...] = (yy + xx.T) - 2.0 * m


def _d2_pallas(y, x):
    My, Nx = y.shape[0], x.shape[0]
    QB = min(My, 256)
    CB = min(Nx, 4096)
    return pl.pallas_call(
        _d2_kernel,
        grid=(My // QB, Nx // CB),
        in_specs=[
            pl.BlockSpec((QB, 3), lambda i, j: (i, 0)),
            pl.BlockSpec((CB, 3), lambda i, j: (j, 0)),
        ],
        out_specs=pl.BlockSpec((QB, CB), lambda i, j: (i, j)),
        out_shape=jax.ShapeDtypeStruct((My, Nx), jnp.float32),
    )(y, x)


def _sc_gather_points(px, py, pz, idx):
    from jax.experimental.pallas import tpu_sc as plsc
    B = idx.shape[0]
    info = plsc.get_sparse_core_info()
    NW = info.num_cores * info.num_subcores
    bpw = B // NW
    mesh = plsc.VectorSubcoreMesh(core_axis_name="c", subcore_axis_name="s")

    @functools.partial(
        pl.kernel, mesh=mesh,
        out_type=[jax.ShapeDtypeStruct((B,), jnp.float32)] * 3,
        scratch_types=[
            pltpu.VMEM((bpw,), jnp.int32),
            pltpu.VMEM((bpw,), jnp.float32),
            pltpu.VMEM((bpw,), jnp.float32),
            pltpu.VMEM((bpw,), jnp.float32),
            pltpu.SemaphoreType.DMA,
        ],
    )
    def k(px_hbm, py_hbm, pz_hbm, idx_hbm, ox_hbm, oy_hbm, oz_hbm,
          idx_v, xv, yv, zv, sem):
        wid = jax.lax.axis_index("s") * info.num_cores + jax.lax.axis_index("c")
        base = wid * bpw
        pltpu.sync_copy(idx_hbm.at[pl.ds(base, bpw)], idx_v)
        pltpu.async_copy(px_hbm.at[idx_v], xv, sem).wait()
        pltpu.async_copy(py_hbm.at[idx_v], yv, sem).wait()
        pltpu.async_copy(pz_hbm.at[idx_v], zv, sem).wait()
        pltpu.sync_copy(xv, ox_hbm.at[pl.ds(base, bpw)])
        pltpu.sync_copy(yv, oy_hbm.at[pl.ds(base, bpw)])
        pltpu.sync_copy(zv, oz_hbm.at[pl.ds(base, bpw)])

    ox, oy, oz = k(px, py, pz, idx)
    return jnp.stack([ox, oy, oz], axis=1)


def _knn_radius(x, y, r, k):
    d2 = (jnp.sum(y * y, axis=1)[:, None] + jnp.sum(x * x, axis=1)[None, :]
          - 2.0 * (y @ x.T))
    neg, idx = jax.lax.top_k(-d2, k)
    valid = (-neg) <= r * r
    return idx, valid


_QB0 = 8
_NR = N // 128


def _knn0_kernel(y_ref, xt_ref, idx_ref, mval_ref, d2_ref):
    y = y_ref[...]
    xt = xt_ref[...]
    yy = y[:, 0:1] * y[:, 0:1] + y[:, 1:2] * y[:, 1:2] + y[:, 2:3] * y[:, 2:3]
    xx = xt[0:1, :] * xt[0:1, :] + xt[1:2, :] * xt[1:2, :] + xt[2:3, :] * xt[2:3, :]
    mm = jnp.dot(y, xt, preferred_element_type=jnp.float32)
    d2 = ((yy + xx) - 2.0 * mm).reshape(_QB0, _NR, 128)
    d2_ref[...] = d2
    bm = jnp.min(d2, axis=2)
    io_bm = jax.lax.broadcasted_iota(jnp.int32, (_QB0, _NR), 1)
    io32 = jax.lax.broadcasted_iota(jnp.int32, (_QB0, K), 1)

    def bstep(kk, carry):
        curv, curi, sb = carry
        elig = (bm > curv) | ((bm == curv) & (io_bm > curi))
        bmm = jnp.where(elig, bm, float("inf"))
        m2 = jnp.min(bmm, axis=1, keepdims=True)
        b2 = jnp.min(jnp.where(elig & (bm == m2), io_bm, 2**30),
                     axis=1, keepdims=True)
        sb = jnp.where(io32 == kk, b2, sb)
        return m2, b2, sb

    _, _, sb = jax.lax.fori_loop(
        0, K, bstep,
        (jnp.full((_QB0, 1), -float("inf"), jnp.float32),
         jnp.full((_QB0, 1), -1, jnp.int32),
         jnp.zeros((_QB0, K), jnp.int32)))

    iofr = (jax.lax.broadcasted_iota(jnp.int32, (_NR, 128), 0) * 128
            + jax.lax.broadcasted_iota(jnp.int32, (_NR, 128), 1)
            ).astype(jnp.float32)
    cds = []
    cis = []
    for q in range(_QB0):
        sel = (sb[q:q + 1, :].reshape(K, 1) ==
               jax.lax.broadcasted_iota(jnp.int32, (K, _NR), 1))
        self32 = sel.astype(jnp.float32)
        cds.append(jnp.dot(self32, d2_ref[q],
                           preferred_element_type=jnp.float32,
                           precision=jax.lax.Precision.HIGHEST))
        cis.append(jnp.dot(self32, iofr,
                           preferred_element_type=jnp.float32,
                           precision=jax.lax.Precision.HIGHEST))
    cd = jnp.stack(cds)
    ci = jnp.stack(cis).astype(jnp.int32)

    def step(kk, carry):
        curv, curi, idxa, mva = carry
        cv = curv[:, :, None]
        cidx = curi[:, :, None]
        elig = (cd > cv) | ((cd == cv) & (ci > cidx))
        cdm = jnp.where(elig, cd, float("inf"))
        m = jnp.min(cdm, axis=(1, 2))[:, None]
        fl = jnp.min(jnp.where(elig & (cd == m[:, :, None]), ci, 2**30),
                     axis=(1, 2))[:, None]
        idxa = jnp.where(io32 == kk, fl, idxa)
        mva = jnp.where(io32 == kk, m, mva)
        return m, fl, idxa, mva

    _, _, idxa, mva = jax.lax.fori_loop(
        0, K, step,
        (jnp.full((_QB0, 1), -float("inf"), jnp.float32),
         jnp.full((_QB0, 1), -1, jnp.int32),
         jnp.zeros((_QB0, K), jnp.int32),
         jnp.zeros((_QB0, K), jnp.float32)))
    idx_ref[...] = idxa
    mval_ref[...] = mva


def _knn0_pallas(points, p1, r):
    xt = points.T
    idx, mval = pl.pallas_call(
        _knn0_kernel,
        grid=(M1 // _QB0,),
        in_specs=[
            pl.BlockSpec((_QB0, 3), lambda i: (i, 0)),
            pl.BlockSpec((3, N), lambda i: (0, 0)),
        ],
        out_specs=[
            pl.BlockSpec((_QB0, K), lambda i: (i, 0)),
            pl.BlockSpec((_QB0, K), lambda i: (i, 0)),
        ],
        out_shape=[
            jax.ShapeDtypeStruct((M1, K), jnp.int32),
            jax.ShapeDtypeStruct((M1, K), jnp.float32),
        ],
        scratch_shapes=[
            pltpu.VMEM((_QB0, _NR, 128), jnp.float32),
        ],
    )(p1, xt)
    valid = mval <= r * r
    return idx, valid


def _decode_kernel(feat_ref, W_ref, b_ref, out2_ref, o_ref):
    d1 = jnp.tanh(feat_ref[...] @ W_ref[...] + b_ref[...])
    o_ref[...] = jnp.tile(out2_ref[...], (1, K)) + d1 * R0


def kernel(points, batch, enc0_W, enc0_b, enc1_W, enc1_b, dec0_W, dec0_b, dec1_W, dec1_b):
    p1, p2 = _fps_pallas(points)
    ptx = points[:, 0].copy()
    pty = points[:, 1].copy()
    ptz = points[:, 2].copy()
    idx0, valid0 = _knn0_pallas(points, p1, R0)
    g0 = _sc_gather_points(ptx, pty, ptz, idx0.reshape(-1)).reshape(M1, K, 3)
    rel0 = jnp.where(valid0[..., None], (g0 - p1[:, None, :]) / R0, 0.0)
    h0 = jax.nn.relu(rel0.reshape(-1, 3) @ enc0_W + enc0_b)
    h0 = jnp.where(valid0.reshape(-1, 1), h0, 0.0)
    f1 = h0.reshape(M1, K, D0).max(axis=1)
    idx1, valid1 = _knn_radius(p1, p2, R1, K)
    rel1 = jnp.where(valid1[..., None], (p1[idx1] - p2[:, None, :]) / R1, 0.0)
    g1 = jnp.where(valid1[..., None], f1[idx1], 0.0)
    h1 = jax.nn.relu(jnp.concatenate([rel1, g1], axis=-1).reshape(-1, 3 + D0) @ enc1_W + enc1_b)
    h1 = jnp.where(valid1.reshape(-1, 1), h1, 0.0)
    f2 = h1.reshape(M2, K, D1).max(axis=1)
    cur = idx1.reshape(-1)
    input_points1 = p1[cur]
    nxt = idx0[cur].reshape(-1)
    input_points0 = _sc_gather_points(ptx, pty, ptz, nxt)
    d0 = (f2 @ dec0_W + dec0_b).reshape(M2, K, 3 + D0)
    rel_a = jnp.tanh(d0[..., :3]).reshape(M2 * K, 3)
    feat_a = jax.nn.relu(d0[..., 3:]).reshape(M2 * K, D0)
    out2 = jnp.repeat(p2, K, axis=0) + rel_a * R1
    out3 = pl.pallas_call(
        _decode_kernel,
        out_shape=jax.ShapeDtypeStruct((M2 * K, K * 3), jnp.float32),
    )(feat_a, dec1_W, dec1_b.reshape(1, K * 3), out2).reshape(M2 * K * K, 3)
    return (out3, f2, input_points0, input_points1)
